# baseline (speedup 1.0000x reference)
import jax
import jax.numpy as jnp
from jax import lax
from jax.experimental import pallas as pl
from jax.experimental.pallas import tpu as pltpu

OH, OW = 3, 3


NG = 11


def _pool_body(b_ref, mask_ref, fm_ref, out_ref):
    del b_ref
    _, C, G, S, W = fm_ref.shape
    H = G * S
    m = mask_ref[0]

    grow = lax.broadcasted_iota(jnp.int32, (G, S, 1), 0)
    srow = lax.broadcasted_iota(jnp.int32, (G, S, 1), 1)
    crow = grow * S + srow
    ccol = lax.broadcasted_iota(jnp.int32, (1, 1, W), 2)
    row_any = jnp.max(m, axis=2, keepdims=True)
    col_any = jnp.max(m, axis=(0, 1), keepdims=True)
    y0 = jnp.min(jnp.where(row_any > 0, crow, H))
    y1 = jnp.max(jnp.where(row_any > 0, crow + 1, 0))
    x0 = jnp.min(jnp.where(col_any > 0, ccol, W))
    x1 = jnp.max(jnp.where(col_any > 0, ccol + 1, 0))
    empty = y1 <= y0
    y0 = jnp.where(empty, 0, y0)
    y1 = jnp.where(empty, H, y1)
    x0 = jnp.where(empty, 0, x0)
    x1 = jnp.where(empty, W, x1)
    h = y1 - y0
    w = x1 - x0

    neg = jnp.float32(-jnp.inf)

    wgrow = lax.broadcasted_iota(jnp.int32, (NG, S, 1), 0)
    wsrow = lax.broadcasted_iota(jnp.int32, (NG, S, 1), 1)
    rowmax = []
    for oy in range(OH):
        sy = y0 + (oy * h) // OH
        ey = y0 + ((oy + 1) * h + OH - 1) // OH
        g0 = jnp.minimum(sy // S, G - NG)
        wrow = (g0 + wgrow) * S + wsrow
        bias = jnp.where((wrow >= sy) & (wrow < ey), 0.0, neg)
        fmw = fm_ref[0, :, pl.ds(g0, NG), :, :]
        mw = mask_ref[0, pl.ds(g0, NG), :, :]
        t = fmw * mw[None] + bias[None]
        rowmax.append(jnp.max(t, axis=(1, 2)))

    ccol2 = lax.broadcasted_iota(jnp.int32, (1, W), 1)
    for ox in range(OW):
        sx = x0 + (ox * w) // OW
        ex = x0 + ((ox + 1) * w + OW - 1) // OW
        cmask = (ccol2 >= sx) & (ccol2 < ex)
        for oy in range(OH):
            red = jnp.max(jnp.where(cmask, rowmax[oy], neg), axis=1)
            out_ref[0, 0, oy * OW + ox, :] = red


def kernel(feature_maps, cell_masks, cell_counts):
    B, C, H, W = feature_maps.shape
    total = cell_masks.shape[0]

    starts = jnp.cumsum(cell_counts.astype(jnp.int32))
    b_for_j = jnp.searchsorted(
        starts, jnp.arange(total, dtype=jnp.int32), side="right"
    ).astype(jnp.int32)

    masks_f = cell_masks.astype(jnp.float32).reshape(total, H // 8, 8, W)
    fm5 = feature_maps.reshape(B, C, H // 8, 8, W)

    CB = 48
    grid_spec = pltpu.PrefetchScalarGridSpec(
        num_scalar_prefetch=1,
        grid=(C // CB, total),
        in_specs=[
            pl.BlockSpec((1, H // 8, 8, W), lambda cb, j, b: (j, 0, 0, 0)),
            pl.BlockSpec((1, CB, H // 8, 8, W), lambda cb, j, b: (b[j], cb, 0, 0, 0)),
        ],
        out_specs=pl.BlockSpec((1, 1, OH * OW, CB), lambda cb, j, b: (j, cb, 0, 0)),
    )

    out = pl.pallas_call(
        _pool_body,
        grid_spec=grid_spec,
        out_shape=jax.ShapeDtypeStruct((total, C // CB, OH * OW, CB), jnp.float32),
        compiler_params=pltpu.CompilerParams(
            dimension_semantics=("arbitrary", "arbitrary"),
        ),
    )(b_for_j, masks_f, fm5)

    return out.transpose(0, 1, 3, 2).reshape(total, C * OH * OW)


# speedup vs baseline: 1.0757x; 1.0757x over previous
import jax
import jax.numpy as jnp
from jax import lax
from jax.experimental import pallas as pl
from jax.experimental.pallas import tpu as pltpu

OH, OW = 3, 3


NG = 11


def _pool_body(b_ref, mask_ref, fm_ref, out_ref):
    del b_ref
    _, C, G, S, W = fm_ref.shape
    H = G * S
    m = mask_ref[0]

    grow = lax.broadcasted_iota(jnp.int32, (G, S, 1), 0)
    srow = lax.broadcasted_iota(jnp.int32, (G, S, 1), 1)
    crow = grow * S + srow
    ccol = lax.broadcasted_iota(jnp.int32, (1, 1, W), 2)
    row_any = jnp.max(m, axis=2, keepdims=True)
    col_any = jnp.max(m, axis=(0, 1), keepdims=True)
    y0 = jnp.min(jnp.where(row_any > 0, crow, H))
    y1 = jnp.max(jnp.where(row_any > 0, crow + 1, 0))
    x0 = jnp.min(jnp.where(col_any > 0, ccol, W))
    x1 = jnp.max(jnp.where(col_any > 0, ccol + 1, 0))
    empty = y1 <= y0
    y0 = jnp.where(empty, 0, y0)
    y1 = jnp.where(empty, H, y1)
    x0 = jnp.where(empty, 0, x0)
    x1 = jnp.where(empty, W, x1)
    h = y1 - y0
    w = x1 - x0

    neg = jnp.float32(-jnp.inf)

    wgrow = lax.broadcasted_iota(jnp.int32, (NG, S, 1), 0)
    wsrow = lax.broadcasted_iota(jnp.int32, (NG, S, 1), 1)
    rowmax = []
    for oy in range(OH):
        sy = y0 + (oy * h) // OH
        ey = y0 + ((oy + 1) * h + OH - 1) // OH
        g0 = jnp.minimum(sy // S, G - NG)
        wrow = (g0 + wgrow) * S + wsrow
        bias = jnp.where((wrow >= sy) & (wrow < ey), 0.0, neg)
        fmw = fm_ref[0, :, pl.ds(g0, NG), :, :]
        mw = mask_ref[0, pl.ds(g0, NG), :, :]
        t = fmw * mw[None] + bias[None]
        rowmax.append(jnp.max(t, axis=(1, 2)))

    ccol2 = lax.broadcasted_iota(jnp.int32, (1, W), 1)
    for ox in range(OW):
        sx = x0 + (ox * w) // OW
        ex = x0 + ((ox + 1) * w + OW - 1) // OW
        cmask = (ccol2 >= sx) & (ccol2 < ex)
        for oy in range(OH):
            red = jnp.max(jnp.where(cmask, rowmax[oy], neg), axis=1)
            out_ref[0, 0, oy * OW + ox, :] = red


def kernel(feature_maps, cell_masks, cell_counts):
    B, C, H, W = feature_maps.shape
    total = cell_masks.shape[0]

    starts = jnp.cumsum(cell_counts.astype(jnp.int32))
    b_for_j = jnp.searchsorted(
        starts, jnp.arange(total, dtype=jnp.int32), side="right"
    ).astype(jnp.int32)

    masks_f = cell_masks.astype(jnp.float32).reshape(total, H // 8, 8, W)
    fm5 = feature_maps.reshape(B, C, H // 8, 8, W)

    CB = 96
    grid_spec = pltpu.PrefetchScalarGridSpec(
        num_scalar_prefetch=1,
        grid=(C // CB, total),
        in_specs=[
            pl.BlockSpec((1, H // 8, 8, W), lambda cb, j, b: (j, 0, 0, 0)),
            pl.BlockSpec((1, CB, H // 8, 8, W), lambda cb, j, b: (b[j], cb, 0, 0, 0)),
        ],
        out_specs=pl.BlockSpec((1, 1, OH * OW, CB), lambda cb, j, b: (j, cb, 0, 0)),
    )

    out = pl.pallas_call(
        _pool_body,
        grid_spec=grid_spec,
        out_shape=jax.ShapeDtypeStruct((total, C // CB, OH * OW, CB), jnp.float32),
        compiler_params=pltpu.CompilerParams(
            dimension_semantics=("arbitrary", "arbitrary"),
        ),
    )(b_for_j, masks_f, fm5)

    return out.transpose(0, 1, 3, 2).reshape(total, C * OH * OW)
